# baseline (speedup 1.0000x reference)
import functools

import jax
import jax.numpy as jnp
from jax import lax
from jax.experimental import pallas as pl
from jax.experimental.pallas import tpu as pltpu
from jax.experimental.pallas import tpu_sc as plsc

_CHUNK = 32
_NBUF = 2


def _sc_body(B, S, D, rows_per_w, table_hbm, out_hbm, bufs, in_sems, out_sems):
    wid = lax.axis_index("s") * 2 + lax.axis_index("c")
    slab = lax.rem(wid, 8)
    quarter = lax.div(wid, 8)
    src_base = slab * 1024 + quarter * rows_per_w
    n_iter = rows_per_w // _CHUNK

    def in_copy(i):
        k = i % _NBUF
        return pltpu.make_async_copy(
            table_hbm.at[pl.ds(src_base + i * _CHUNK, _CHUNK), :],
            bufs.at[k],
            in_sems.at[k],
        )

    def out_copy(i, b):
        k = i % _NBUF
        return pltpu.make_async_copy(
            bufs.at[k],
            out_hbm.at[pl.ds(b * S + src_base + i * _CHUNK, _CHUNK), :],
            out_sems.at[k],
        )

    in_copy(0).start()
    for i in range(n_iter):
        if i + 1 < n_iter:
            if i + 1 >= _NBUF:
                for b in range(B):
                    out_copy(i + 1 - _NBUF, b).wait()
            in_copy(i + 1).start()
        in_copy(i).wait()
        for b in range(B):
            out_copy(i, b).start()
    for i in range(max(0, n_iter - _NBUF), n_iter):
        for b in range(B):
            out_copy(i, b).wait()


def kernel(x, pos_embeddings):
    B, S = x.shape
    D = pos_embeddings.shape[1]
    rows_per_w = S // 8 // 4
    mesh = plsc.VectorSubcoreMesh(core_axis_name="c", subcore_axis_name="s")
    k = pl.kernel(
        functools.partial(_sc_body, B, S, D, rows_per_w),
        out_type=jax.ShapeDtypeStruct((B * S, D), pos_embeddings.dtype),
        mesh=mesh,
        scratch_types=[
            pltpu.VMEM((_NBUF, _CHUNK, D), pos_embeddings.dtype),
            pltpu.SemaphoreType.DMA((_NBUF,)),
            pltpu.SemaphoreType.DMA((_NBUF,)),
        ],
    )
    out_flat = k(pos_embeddings)
    return out_flat.reshape(B, S, D)


# speedup vs baseline: 1.0142x; 1.0142x over previous
import functools

import jax
import jax.numpy as jnp
from jax import lax
from jax.experimental import pallas as pl
from jax.experimental.pallas import tpu as pltpu
from jax.experimental.pallas import tpu_sc as plsc

_CHUNK = 32
_NBUF = 3


def _sc_body(B, S, D, rows_per_w, table_hbm, out_hbm, bufs, in_sems, out_sems):
    wid = lax.axis_index("s") * 2 + lax.axis_index("c")
    slab = lax.rem(wid, 8)
    quarter = lax.div(wid, 8)
    src_base = slab * 1024 + quarter * rows_per_w
    n_iter = rows_per_w // _CHUNK

    def in_copy(i):
        k = i % _NBUF
        return pltpu.make_async_copy(
            table_hbm.at[pl.ds(src_base + i * _CHUNK, _CHUNK), :],
            bufs.at[k],
            in_sems.at[k],
        )

    def out_copy(i, b):
        k = i % _NBUF
        return pltpu.make_async_copy(
            bufs.at[k],
            out_hbm.at[pl.ds(b * S + src_base + i * _CHUNK, _CHUNK), :],
            out_sems.at[k],
        )

    in_copy(0).start()
    for i in range(n_iter):
        if i + 1 < n_iter:
            if i + 1 >= _NBUF:
                for b in range(B):
                    out_copy(i + 1 - _NBUF, b).wait()
            in_copy(i + 1).start()
        in_copy(i).wait()
        for b in range(B):
            out_copy(i, b).start()
    for i in range(max(0, n_iter - _NBUF), n_iter):
        for b in range(B):
            out_copy(i, b).wait()


def kernel(x, pos_embeddings):
    B, S = x.shape
    D = pos_embeddings.shape[1]
    rows_per_w = S // 8 // 4
    mesh = plsc.VectorSubcoreMesh(core_axis_name="c", subcore_axis_name="s")
    k = pl.kernel(
        functools.partial(_sc_body, B, S, D, rows_per_w),
        out_type=jax.ShapeDtypeStruct((B * S, D), pos_embeddings.dtype),
        mesh=mesh,
        scratch_types=[
            pltpu.VMEM((_NBUF, _CHUNK, D), pos_embeddings.dtype),
            pltpu.SemaphoreType.DMA((_NBUF,)),
            pltpu.SemaphoreType.DMA((_NBUF,)),
        ],
    )
    out_flat = k(pos_embeddings)
    return out_flat.reshape(B, S, D)


# speedup vs baseline: 1.0422x; 1.0275x over previous
import functools

import jax
import jax.numpy as jnp
from jax import lax
from jax.experimental import pallas as pl
from jax.experimental.pallas import tpu as pltpu
from jax.experimental.pallas import tpu_sc as plsc

_SIZES = (56, 56, 56, 56, 32)
_OFFS = (0, 56, 112, 168, 224)
_NBUF = 2
_BUF_ROWS = max(_SIZES)


def _sc_body(B, S, D, rows_per_w, table_hbm, out_hbm, bufs, in_sems, out_sems):
    wid = lax.axis_index("s") * 2 + lax.axis_index("c")
    slab = lax.rem(wid, 8)
    quarter = lax.div(wid, 8)
    src_base = slab * 1024 + quarter * rows_per_w
    n_iter = len(_SIZES)

    def in_copy(i):
        k = i % _NBUF
        return pltpu.make_async_copy(
            table_hbm.at[pl.ds(src_base + _OFFS[i], _SIZES[i]), :],
            bufs.at[k, pl.ds(0, _SIZES[i]), :],
            in_sems.at[k],
        )

    def out_copy(i, b):
        k = i % _NBUF
        return pltpu.make_async_copy(
            bufs.at[k, pl.ds(0, _SIZES[i]), :],
            out_hbm.at[pl.ds(b * S + src_base + _OFFS[i], _SIZES[i]), :],
            out_sems.at[k],
        )

    in_copy(0).start()
    for i in range(n_iter):
        if i + 1 < n_iter:
            if i + 1 >= _NBUF:
                for b in range(B):
                    out_copy(i + 1 - _NBUF, b).wait()
            in_copy(i + 1).start()
        in_copy(i).wait()
        for b in range(B):
            out_copy(i, b).start()
    for i in range(max(0, n_iter - _NBUF), n_iter):
        for b in range(B):
            out_copy(i, b).wait()


def kernel(x, pos_embeddings):
    B, S = x.shape
    D = pos_embeddings.shape[1]
    rows_per_w = S // 8 // 4
    mesh = plsc.VectorSubcoreMesh(core_axis_name="c", subcore_axis_name="s")
    k = pl.kernel(
        functools.partial(_sc_body, B, S, D, rows_per_w),
        out_type=jax.ShapeDtypeStruct((B * S, D), pos_embeddings.dtype),
        mesh=mesh,
        scratch_types=[
            pltpu.VMEM((_NBUF, _BUF_ROWS, D), pos_embeddings.dtype),
            pltpu.SemaphoreType.DMA((_NBUF,)),
            pltpu.SemaphoreType.DMA((_NBUF,)),
        ],
    )
    out_flat = k(pos_embeddings)
    return out_flat.reshape(B, S, D)
